# baseline (speedup 1.0000x reference)
import jax
import jax.numpy as jnp
from jax.experimental import pallas as pl
from jax.experimental.pallas import tpu as pltpu

B, S, C, D = 1024, 200, 32, 32
NIH, NAH, NUH = 2, 2, 4

BB = 16


def _fused_kernel(u0_ref, uemb_ref, h0_ref, hp_ref, ha_ref, hact_ref, hprod_ref,
                  c0_ref, cp_ref, ca_ref, cprod_ref,
                  w1_ref, w2_ref, w3_ref,
                  cand_out, cand_mask, hist_out, hist_mask, user_out, user_mask):
    f32 = jnp.float32
    user_out[...] = jnp.dot(uemb_ref[...], w1_ref[...], preferred_element_type=f32)
    user_mask[...] = u0_ref[...] != 0
    w2 = w2_ref[...]
    acc_c = jnp.dot(cp_ref[...].reshape(BB * C, NIH * D), w2[: NIH * D],
                    preferred_element_type=f32)
    acc_c += jnp.dot(ca_ref[...].reshape(BB * C, NAH * D),
                     w2[NIH * D:(NIH + NAH) * D], preferred_element_type=f32)
    acc_c += jnp.dot(cprod_ref[...].reshape(BB * C, D),
                     w2[(NIH + NAH) * D:], preferred_element_type=f32)
    cand_out[...] = acc_c.reshape(BB, C, D)
    cand_mask[...] = c0_ref[...] != 0
    w3 = w3_ref[...]
    acc_h = jnp.dot(hp_ref[...].reshape(BB * S, NIH * D), w3[: NIH * D],
                    preferred_element_type=f32)
    acc_h += jnp.dot(ha_ref[...].reshape(BB * S, NAH * D),
                     w3[NIH * D:(NIH + NAH) * D], preferred_element_type=f32)
    acc_h += jnp.dot(hact_ref[...].reshape(BB * S, D),
                     w3[(NIH + NAH) * D:(NIH + NAH + 1) * D],
                     preferred_element_type=f32)
    acc_h += jnp.dot(hprod_ref[...].reshape(BB * S, D),
                     w3[(NIH + NAH + 1) * D:], preferred_element_type=f32)
    hist_out[...] = acc_h.reshape(BB, S, D)
    hist_mask[...] = h0_ref[...] != 0


def kernel(user_hashes, user_embeddings, history_post_hashes, history_post_embeddings,
           history_author_embeddings, history_product_surface_embeddings,
           history_actions_embeddings, candidate_post_hashes, candidate_post_embeddings,
           candidate_author_embeddings, candidate_product_surface_embeddings,
           proj_mat_1, proj_mat_2, proj_mat_3):
    u0 = user_hashes[:, :1].astype(jnp.int32)
    h0 = history_post_hashes[:, :, 0].astype(jnp.int32)
    c0 = candidate_post_hashes[:, :, 0].astype(jnp.int32)
    uemb = user_embeddings.reshape(B, NUH * D)
    hp = history_post_embeddings.reshape(B, S, NIH * D)
    ha = history_author_embeddings.reshape(B, S, NAH * D)
    cp = candidate_post_embeddings.reshape(B, C, NIH * D)
    ca = candidate_author_embeddings.reshape(B, C, NAH * D)

    grid = (B // BB,)
    bspec = lambda *blk: pl.BlockSpec(blk, lambda i: (i,) + (0,) * (len(blk) - 1))
    wspec = lambda *blk: pl.BlockSpec(blk, lambda i: (0,) * len(blk))

    out_shapes = (
        jax.ShapeDtypeStruct((B, C, D), jnp.float32),
        jax.ShapeDtypeStruct((B, C), jnp.bool_),
        jax.ShapeDtypeStruct((B, S, D), jnp.float32),
        jax.ShapeDtypeStruct((B, S), jnp.bool_),
        jax.ShapeDtypeStruct((B, D), jnp.float32),
        jax.ShapeDtypeStruct((B, 1), jnp.bool_),
    )
    out_specs = (
        bspec(BB, C, D),
        bspec(BB, C),
        bspec(BB, S, D),
        bspec(BB, S),
        bspec(BB, D),
        bspec(BB, 1),
    )
    in_specs = (
        bspec(BB, 1),
        bspec(BB, NUH * D),
        bspec(BB, S),
        bspec(BB, S, NIH * D),
        bspec(BB, S, NAH * D),
        bspec(BB, S, D),
        bspec(BB, S, D),
        bspec(BB, C),
        bspec(BB, C, NIH * D),
        bspec(BB, C, NAH * D),
        bspec(BB, C, D),
        wspec(NUH * D, D),
        wspec((NIH + NAH + 1) * D, D),
        wspec((NIH + NAH + 2) * D, D),
    )

    cand_emb, cand_mask, hist_emb, hist_mask, user_emb, user_mask = pl.pallas_call(
        _fused_kernel,
        grid=grid,
        in_specs=in_specs,
        out_specs=out_specs,
        out_shape=out_shapes,
    )(u0, uemb, h0, hp, ha, history_actions_embeddings,
      history_product_surface_embeddings, c0, cp, ca,
      candidate_product_surface_embeddings, proj_mat_1, proj_mat_2, proj_mat_3)

    return (cand_emb, cand_mask, hist_emb, hist_mask,
            user_emb.reshape(B, 1, D), user_mask)


# speedup vs baseline: 1.0057x; 1.0057x over previous
import jax
import jax.numpy as jnp
from jax.experimental import pallas as pl
from jax.experimental.pallas import tpu as pltpu

B, S, C, D = 1024, 200, 32, 32
NIH, NAH, NUH = 2, 2, 4

BB = 32


def _fused_kernel(u0_ref, uemb_ref, h0_ref, hp_ref, ha_ref, hact_ref, hprod_ref,
                  c0_ref, cp_ref, ca_ref, cprod_ref,
                  w1_ref, w2_ref, w3_ref,
                  cand_out, cand_mask, hist_out, hist_mask, user_out, user_mask):
    f32 = jnp.float32
    user_out[...] = jnp.dot(uemb_ref[...], w1_ref[...], preferred_element_type=f32)
    user_mask[...] = u0_ref[...] != 0
    w2 = w2_ref[...]
    acc_c = jnp.dot(cp_ref[...].reshape(BB * C, NIH * D), w2[: NIH * D],
                    preferred_element_type=f32)
    acc_c += jnp.dot(ca_ref[...].reshape(BB * C, NAH * D),
                     w2[NIH * D:(NIH + NAH) * D], preferred_element_type=f32)
    acc_c += jnp.dot(cprod_ref[...].reshape(BB * C, D),
                     w2[(NIH + NAH) * D:], preferred_element_type=f32)
    cand_out[...] = acc_c.reshape(BB, C, D)
    cand_mask[...] = c0_ref[...] != 0
    w3 = w3_ref[...]
    acc_h = jnp.dot(hp_ref[...].reshape(BB * S, NIH * D), w3[: NIH * D],
                    preferred_element_type=f32)
    acc_h += jnp.dot(ha_ref[...].reshape(BB * S, NAH * D),
                     w3[NIH * D:(NIH + NAH) * D], preferred_element_type=f32)
    acc_h += jnp.dot(hact_ref[...].reshape(BB * S, D),
                     w3[(NIH + NAH) * D:(NIH + NAH + 1) * D],
                     preferred_element_type=f32)
    acc_h += jnp.dot(hprod_ref[...].reshape(BB * S, D),
                     w3[(NIH + NAH + 1) * D:], preferred_element_type=f32)
    hist_out[...] = acc_h.reshape(BB, S, D)
    hist_mask[...] = h0_ref[...] != 0


def kernel(user_hashes, user_embeddings, history_post_hashes, history_post_embeddings,
           history_author_embeddings, history_product_surface_embeddings,
           history_actions_embeddings, candidate_post_hashes, candidate_post_embeddings,
           candidate_author_embeddings, candidate_product_surface_embeddings,
           proj_mat_1, proj_mat_2, proj_mat_3):
    u0 = user_hashes[:, :1].astype(jnp.int32)
    h0 = history_post_hashes[:, :, 0].astype(jnp.int32)
    c0 = candidate_post_hashes[:, :, 0].astype(jnp.int32)
    uemb = user_embeddings.reshape(B, NUH * D)
    hp = history_post_embeddings.reshape(B, S, NIH * D)
    ha = history_author_embeddings.reshape(B, S, NAH * D)
    cp = candidate_post_embeddings.reshape(B, C, NIH * D)
    ca = candidate_author_embeddings.reshape(B, C, NAH * D)

    grid = (B // BB,)
    bspec = lambda *blk: pl.BlockSpec(blk, lambda i: (i,) + (0,) * (len(blk) - 1))
    wspec = lambda *blk: pl.BlockSpec(blk, lambda i: (0,) * len(blk))

    out_shapes = (
        jax.ShapeDtypeStruct((B, C, D), jnp.float32),
        jax.ShapeDtypeStruct((B, C), jnp.bool_),
        jax.ShapeDtypeStruct((B, S, D), jnp.float32),
        jax.ShapeDtypeStruct((B, S), jnp.bool_),
        jax.ShapeDtypeStruct((B, D), jnp.float32),
        jax.ShapeDtypeStruct((B, 1), jnp.bool_),
    )
    out_specs = (
        bspec(BB, C, D),
        bspec(BB, C),
        bspec(BB, S, D),
        bspec(BB, S),
        bspec(BB, D),
        bspec(BB, 1),
    )
    in_specs = (
        bspec(BB, 1),
        bspec(BB, NUH * D),
        bspec(BB, S),
        bspec(BB, S, NIH * D),
        bspec(BB, S, NAH * D),
        bspec(BB, S, D),
        bspec(BB, S, D),
        bspec(BB, C),
        bspec(BB, C, NIH * D),
        bspec(BB, C, NAH * D),
        bspec(BB, C, D),
        wspec(NUH * D, D),
        wspec((NIH + NAH + 1) * D, D),
        wspec((NIH + NAH + 2) * D, D),
    )

    cand_emb, cand_mask, hist_emb, hist_mask, user_emb, user_mask = pl.pallas_call(
        _fused_kernel,
        grid=grid,
        in_specs=in_specs,
        out_specs=out_specs,
        out_shape=out_shapes,
    )(u0, uemb, h0, hp, ha, history_actions_embeddings,
      history_product_surface_embeddings, c0, cp, ca,
      candidate_product_surface_embeddings, proj_mat_1, proj_mat_2, proj_mat_3)

    return (cand_emb, cand_mask, hist_emb, hist_mask,
            user_emb.reshape(B, 1, D), user_mask)
